# baseline (speedup 1.0000x reference)
import functools

import jax
import jax.numpy as jnp
import numpy as np
from jax import lax
from jax.experimental import pallas as pl
from jax.experimental.pallas import tpu as pltpu
from jax.experimental.pallas import tpu_sc as plsc

NC = 2
NS = 16
L = 16
NW = NC * NS

D = 128
NN = 32
G = 4
ROWS = G * NN
NBUF = 4


def _gather_mean(x, xbf, nodes3, neigh3, b_pad):
    b_per_w = b_pad // NW
    n_groups = b_per_w // G
    n_outer = n_groups // NBUF
    assert n_outer * NBUF == n_groups
    mesh = plsc.VectorSubcoreMesh(core_axis_name="c", subcore_axis_name="s")

    @functools.partial(
        pl.kernel,
        mesh=mesh,
        compiler_params=pltpu.CompilerParams(use_tc_tiling_on_sc=False),
        out_type=jax.ShapeDtypeStruct((b_pad, 2 * D), jnp.float32),
        scratch_types=[
            pltpu.VMEM((n_groups, ROWS), jnp.int32),
            pltpu.VMEM((b_per_w,), jnp.int32),
            pltpu.VMEM((b_per_w, D), jnp.float32),
            pltpu.VMEM((NBUF, ROWS, D // 2), jnp.int32),
            pltpu.VMEM((NBUF, G, D), jnp.float32),
            pltpu.SemaphoreType.DMA((NBUF,)),
            pltpu.SemaphoreType.DMA((NBUF,)),
            pltpu.SemaphoreType.DMA,
        ],
    )
    def k(x_hbm, xbf_hbm, nodes_hbm, neigh_hbm, h_hbm,
          nidx_v, sidx_v, sbuf, nbuf, hbuf, gsem, osem, ssem):
        wid = lax.axis_index("s") * NC + lax.axis_index("c")
        base_row = wid * b_per_w
        pltpu.sync_copy(neigh_hbm.at[wid], nidx_v)
        pltpu.sync_copy(nodes_hbm.at[wid], sidx_v)
        for lo in range(0, b_per_w, 128):
            sz = min(128, b_per_w - lo)
            pltpu.async_copy(
                x_hbm.at[sidx_v.at[pl.ds(lo, sz)]], sbuf.at[pl.ds(lo, sz)], ssem
            ).wait()
        pltpu.sync_copy(sbuf, h_hbm.at[pl.ds(base_row, b_per_w), pl.ds(0, D)])

        def gather(g, slot):
            return pltpu.make_async_copy(
                xbf_hbm.at[nidx_v.at[g]], nbuf.at[slot], gsem.at[slot]
            )

        def agg_write(g, slot):
            return pltpu.make_async_copy(
                hbuf.at[slot],
                h_hbm.at[pl.ds(base_row + g * G, G), pl.ds(D, D)],
                osem.at[slot],
            )

        for slot in range(NBUF):
            gather(slot, slot).start()

        def outer(go, carry):
            for slot in range(NBUF):
                g = go * NBUF + slot
                gather(g, slot).wait()
                @pl.when(go > 0)
                def _():
                    agg_write(g - NBUF, slot).wait()
                for si in range(G):
                    UNR = 8
                    NCH = D // (2 * L)
                    hi_msk = jnp.int32(-65536)

                    def body(t, accs):
                        lo_a, hi_a = accs
                        row0 = si * NN + t * UNR
                        for u in range(UNR):
                            for ci in range(NCH):
                                w = nbuf[slot, row0 + u, pl.ds(ci * L, L)]
                                hi = lax.bitcast_convert_type(w & hi_msk, jnp.float32)
                                lo = lax.bitcast_convert_type(w << 16, jnp.float32)
                                lo_a = lo_a[:ci] + (lo_a[ci] + lo,) + lo_a[ci + 1:]
                                hi_a = hi_a[:ci] + (hi_a[ci] + hi,) + hi_a[ci + 1:]
                        return (lo_a, hi_a)

                    zero = tuple(jnp.zeros((L,), jnp.float32) for _ in range(NCH))
                    lo_a, hi_a = lax.fori_loop(0, NN // UNR, body, (zero, zero))
                    for ci in range(NCH):
                        hbuf[slot, si, pl.ds(ci * 2 * L, L)] = lo_a[ci]
                        hbuf[slot, si, pl.ds(ci * 2 * L + L, L)] = hi_a[ci]
                agg_write(g, slot).start()
                @pl.when(go < n_outer - 1)
                def _():
                    gather(g + NBUF, slot).start()
            return carry

        lax.fori_loop(0, n_outer, outer, 0)
        for slot in range(NBUF):
            agg_write((n_outer - 1) * NBUF + slot, slot).wait()

    return k(x, xbf, nodes3, neigh3)


def _mm_body(h_ref, w_ref, b_ref, o_ref):
    o_ref[...] = (
        lax.dot_general(
            h_ref[...],
            w_ref[...],
            (((1,), (0,)), ((), ())),
            preferred_element_type=jnp.float32,
        )
        + b_ref[...]
    )


def _linear(h, W, b, n_out):
    b_pad = h.shape[0]
    blk = 1024
    grid = b_pad // blk
    return pl.pallas_call(
        _mm_body,
        grid=(grid,),
        in_specs=[
            pl.BlockSpec((blk, 2 * D), lambda i: (i, 0)),
            pl.BlockSpec((2 * D, D), lambda i: (0, 0)),
            pl.BlockSpec((1, D), lambda i: (0, 0)),
        ],
        out_specs=pl.BlockSpec((blk, D), lambda i: (i, 0)),
        out_shape=jax.ShapeDtypeStruct((n_out, D), jnp.float32),
    )(h, W, b.reshape(1, D))


def kernel(x, nodes, neigh_idx, W, b):
    B, n_neigh = neigh_idx.shape
    assert n_neigh == NN and x.shape[1] == D
    chunk = 1024
    b_pad = ((B + chunk - 1) // chunk) * chunk
    pad = b_pad - B
    nodes_p = jnp.concatenate([nodes, jnp.zeros((pad,), jnp.int32)])
    neigh_p = jnp.concatenate([neigh_idx, jnp.zeros((pad, NN), jnp.int32)], axis=0)
    b_per_w = b_pad // NW
    nodes3 = nodes_p.reshape(NW, b_per_w)
    neigh3 = neigh_p.reshape(NW, b_per_w // G, G * NN)
    xbf = lax.bitcast_convert_type(
        x.astype(jnp.bfloat16).reshape(x.shape[0], D // 2, 2), jnp.int32
    )
    h = _gather_mean(x, xbf, nodes3, neigh3, b_pad)
    perm = np.arange(D).reshape(D // 32, 16, 2).transpose(0, 2, 1).reshape(D)
    W_bot = (W[D:] * jnp.float32(1.0 / NN))[perm]
    W_adj = jnp.concatenate([W[:D], W_bot], axis=0)
    return _linear(h, W_adj, b, B)


# speedup vs baseline: 1.2698x; 1.2698x over previous
import functools

import jax
import jax.numpy as jnp
from jax import lax
from jax.experimental import pallas as pl
from jax.experimental.pallas import tpu as pltpu
from jax.experimental.pallas import tpu_sc as plsc

NC = 2
NS = 16
L = 16
NW = NC * NS

D = 128
NN = 32
G = 4
ROWS = G * NN
NBUF = 4


def _gather_mean(x, nodes3, neigh3, b_pad):
    b_per_w = b_pad // NW
    n_groups = b_per_w // G
    n_outer = n_groups // NBUF
    assert n_outer * NBUF == n_groups
    mesh = plsc.VectorSubcoreMesh(core_axis_name="c", subcore_axis_name="s")

    @functools.partial(
        pl.kernel,
        mesh=mesh,
        out_type=[
            jax.ShapeDtypeStruct((b_pad, D), jnp.float32),
            jax.ShapeDtypeStruct((b_pad, D), jnp.float32),
        ],
        scratch_types=[
            pltpu.VMEM((n_groups, ROWS), jnp.int32),
            pltpu.VMEM((b_per_w,), jnp.int32),
            pltpu.VMEM((NBUF, ROWS, D), jnp.float32),
            pltpu.VMEM((NBUF, G, D), jnp.float32),
            pltpu.SemaphoreType.DMA((NBUF,)),
            pltpu.SemaphoreType.DMA((NBUF,)),
        ],
    )
    def k(x_hbm, nodes_hbm, neigh_hbm, hs_hbm, ha_hbm, nidx_v, sidx_v, nbuf, hbuf, gsem, osem):
        wid = lax.axis_index("s") * NC + lax.axis_index("c")
        base_row = wid * b_per_w
        pltpu.sync_copy(neigh_hbm.at[wid], nidx_v)
        pltpu.sync_copy(nodes_hbm.at[wid], sidx_v)
        for j, lo in enumerate(range(0, b_per_w, 128)):
            sz = min(128, b_per_w - lo)
            pltpu.async_copy(
                x_hbm.at[sidx_v.at[pl.ds(lo, sz)]],
                nbuf.at[j % NBUF, pl.ds(0, sz)],
                gsem.at[j % NBUF],
            ).wait()
            pltpu.sync_copy(
                nbuf.at[j % NBUF, pl.ds(0, sz)],
                hs_hbm.at[pl.ds(base_row + lo, sz)],
            )

        def gather(g, slot):
            return pltpu.make_async_copy(
                x_hbm.at[nidx_v.at[g]], nbuf.at[slot], gsem.at[slot]
            )

        def agg_write(g, slot):
            return pltpu.make_async_copy(
                hbuf.at[slot],
                ha_hbm.at[pl.ds(base_row + g * G, G)],
                osem.at[slot],
            )

        for slot in range(NBUF):
            gather(slot, slot).start()

        def outer(go, carry):
            for slot in range(NBUF):
                g = go * NBUF + slot
                gather(g, slot).wait()
                @pl.when(go > 0)
                def _():
                    agg_write(g - NBUF, slot).wait()
                for si in range(G):
                    UNR = 8

                    def body(t, accs):
                        row0 = si * NN + t * UNR
                        for u in range(UNR):
                            accs = tuple(
                                accs[ci] + nbuf[slot, row0 + u, pl.ds(ci * L, L)]
                                for ci in range(D // L)
                            )
                        return accs

                    accs = lax.fori_loop(
                        0, NN // UNR, body,
                        tuple(jnp.zeros((L,), jnp.float32) for _ in range(D // L)),
                    )
                    for ci in range(D // L):
                        hbuf[slot, si, pl.ds(ci * L, L)] = accs[ci]
                agg_write(g, slot).start()
                @pl.when(go < n_outer - 1)
                def _():
                    gather(g + NBUF, slot).start()
            return carry

        lax.fori_loop(0, n_outer, outer, 0)
        for slot in range(NBUF):
            agg_write((n_outer - 1) * NBUF + slot, slot).wait()

    return k(x, nodes3, neigh3)


def _mm_body(hs_ref, ha_ref, wt_ref, wb_ref, b_ref, o_ref):
    dims = (((1,), (0,)), ((), ()))
    o_ref[...] = (
        lax.dot_general(hs_ref[...], wt_ref[...], dims,
                        preferred_element_type=jnp.float32)
        + lax.dot_general(ha_ref[...], wb_ref[...], dims,
                          preferred_element_type=jnp.float32)
        + b_ref[...]
    )


def _linear(hs, ha, W_top, W_bot, b, n_out):
    b_pad = hs.shape[0]
    blk = 1024
    grid = b_pad // blk
    return pl.pallas_call(
        _mm_body,
        grid=(grid,),
        in_specs=[
            pl.BlockSpec((blk, D), lambda i: (i, 0)),
            pl.BlockSpec((blk, D), lambda i: (i, 0)),
            pl.BlockSpec((D, D), lambda i: (0, 0)),
            pl.BlockSpec((D, D), lambda i: (0, 0)),
            pl.BlockSpec((1, D), lambda i: (0, 0)),
        ],
        out_specs=pl.BlockSpec((blk, D), lambda i: (i, 0)),
        out_shape=jax.ShapeDtypeStruct((n_out, D), jnp.float32),
    )(hs, ha, W_top, W_bot, b.reshape(1, D))


def kernel(x, nodes, neigh_idx, W, b):
    B, n_neigh = neigh_idx.shape
    assert n_neigh == NN and x.shape[1] == D
    chunk = 1024
    b_pad = ((B + chunk - 1) // chunk) * chunk
    pad = b_pad - B
    nodes_p = jnp.concatenate([nodes, jnp.zeros((pad,), jnp.int32)])
    neigh_p = jnp.concatenate([neigh_idx, jnp.zeros((pad, NN), jnp.int32)], axis=0)
    b_per_w = b_pad // NW
    nodes3 = nodes_p.reshape(NW, b_per_w)
    neigh3 = neigh_p.reshape(NW, b_per_w // G, G * NN)
    hs, ha = _gather_mean(x, nodes3, neigh3, b_pad)
    return _linear(hs, ha, W[:D], W[D:] * jnp.float32(1.0 / NN), b, B)


# speedup vs baseline: 1.7904x; 1.4099x over previous
import functools

import jax
import jax.numpy as jnp
from jax import lax
from jax.experimental import pallas as pl
from jax.experimental.pallas import tpu as pltpu
from jax.experimental.pallas import tpu_sc as plsc

NC = 2
NS = 16
L = 16

D = 128
NN = 32
G = 4
ROWS = G * NN
NBUF = 4

FAST_C = 0
F_SEEDS = 576
S_SEEDS = 64
B_PAD = NS * (F_SEEDS + S_SEEDS)


def _gather_mean(x, nodes_p, neigh2):
    fg = F_SEEDS // G
    sg = S_SEEDS // G
    mesh = plsc.VectorSubcoreMesh(core_axis_name="c", subcore_axis_name="s")

    @functools.partial(
        pl.kernel,
        mesh=mesh,
        out_type=[
            jax.ShapeDtypeStruct((B_PAD, D), jnp.float32),
            jax.ShapeDtypeStruct((B_PAD, D), jnp.float32),
        ],
        scratch_types=[
            pltpu.VMEM((fg, ROWS), jnp.int32),
            pltpu.VMEM((F_SEEDS,), jnp.int32),
            pltpu.VMEM((128, D), jnp.float32),
            pltpu.VMEM((NBUF, ROWS, D), jnp.float32),
            pltpu.VMEM((NBUF, G, D), jnp.float32),
            pltpu.SemaphoreType.DMA((NBUF,)),
            pltpu.SemaphoreType.DMA((NBUF,)),
            pltpu.SemaphoreType.DMA,
        ],
    )
    def k(x_hbm, nodes_hbm, neigh_hbm, hs_hbm, ha_hbm,
          nidx_v, sidx_v, sbuf, nbuf, hbuf, gsem, osem, ssem):
        c = lax.axis_index("c")
        s = lax.axis_index("s")
        on_fast = c == FAST_C
        my_seeds = jnp.where(on_fast, F_SEEDS, S_SEEDS)
        base_row = jnp.where(on_fast, s * F_SEEDS, NS * F_SEEDS + s * S_SEEDS)
        base_row = pl.multiple_of(base_row, 64)
        n_groups = jnp.where(on_fast, fg, sg)
        gbase = base_row // G
        gbase = pl.multiple_of(gbase, 8)

        @pl.when(on_fast)
        def _():
            pltpu.sync_copy(neigh_hbm.at[pl.ds(gbase, fg)], nidx_v)
            pltpu.sync_copy(nodes_hbm.at[pl.ds(base_row, F_SEEDS)], sidx_v)

        @pl.when(jnp.logical_not(on_fast))
        def _():
            pltpu.sync_copy(
                neigh_hbm.at[pl.ds(gbase, sg)], nidx_v.at[pl.ds(0, sg)]
            )
            pltpu.sync_copy(
                nodes_hbm.at[pl.ds(base_row, S_SEEDS)],
                sidx_v.at[pl.ds(0, S_SEEDS)],
            )

        def self_chunk(lo, sz):
            pltpu.async_copy(
                x_hbm.at[sidx_v.at[pl.ds(lo, sz)]], sbuf.at[pl.ds(0, sz)], ssem
            ).wait()
            pltpu.sync_copy(
                sbuf.at[pl.ds(0, sz)], hs_hbm.at[pl.ds(base_row + lo, sz)]
            )

        def self_loop(i, carry):
            self_chunk(i * 128, 128)
            return carry

        n_full = my_seeds // 128
        lax.fori_loop(0, n_full, self_loop, 0)
        rem_lo = n_full * 128

        @pl.when(on_fast)
        def _():
            self_chunk(rem_lo, F_SEEDS % 128)

        @pl.when(jnp.logical_not(on_fast))
        def _():
            self_chunk(rem_lo, S_SEEDS)

        def gather(g, slot):
            return pltpu.make_async_copy(
                x_hbm.at[nidx_v.at[g]], nbuf.at[slot], gsem.at[slot]
            )

        def agg_write(g, slot):
            return pltpu.make_async_copy(
                hbuf.at[slot],
                ha_hbm.at[pl.ds(base_row + g * G, G)],
                osem.at[slot],
            )

        for slot in range(NBUF):
            gather(slot, slot).start()

        n_outer = n_groups // NBUF

        def outer(go, carry):
            for slot in range(NBUF):
                g = go * NBUF + slot
                gather(g, slot).wait()
                @pl.when(go > 0)
                def _():
                    agg_write(g - NBUF, slot).wait()
                for si in range(G):
                    UNR = 8

                    def body(t, accs):
                        row0 = si * NN + t * UNR
                        for u in range(UNR):
                            accs = tuple(
                                accs[ci] + nbuf[slot, row0 + u, pl.ds(ci * L, L)]
                                for ci in range(D // L)
                            )
                        return accs

                    accs = lax.fori_loop(
                        0, NN // UNR, body,
                        tuple(jnp.zeros((L,), jnp.float32) for _ in range(D // L)),
                    )
                    for ci in range(D // L):
                        hbuf[slot, si, pl.ds(ci * L, L)] = accs[ci]
                agg_write(g, slot).start()
                @pl.when(go < n_outer - 1)
                def _():
                    gather(g + NBUF, slot).start()
            return carry

        lax.fori_loop(0, n_outer, outer, 0)
        for slot in range(NBUF):
            agg_write((n_outer - 1) * NBUF + slot, slot).wait()

    return k(x, nodes_p, neigh2)


def _mm_body(hs_ref, ha_ref, wt_ref, wb_ref, b_ref, o_ref):
    dims = (((1,), (0,)), ((), ()))
    o_ref[...] = (
        lax.dot_general(hs_ref[...], wt_ref[...], dims,
                        preferred_element_type=jnp.float32)
        + lax.dot_general(ha_ref[...], wb_ref[...], dims,
                          preferred_element_type=jnp.float32)
        + b_ref[...]
    )


def _linear(hs, ha, W_top, W_bot, b, n_out):
    blk = 1024
    grid = hs.shape[0] // blk
    return pl.pallas_call(
        _mm_body,
        grid=(grid,),
        in_specs=[
            pl.BlockSpec((blk, D), lambda i: (i, 0)),
            pl.BlockSpec((blk, D), lambda i: (i, 0)),
            pl.BlockSpec((D, D), lambda i: (0, 0)),
            pl.BlockSpec((D, D), lambda i: (0, 0)),
            pl.BlockSpec((1, D), lambda i: (0, 0)),
        ],
        out_specs=pl.BlockSpec((blk, D), lambda i: (i, 0)),
        out_shape=jax.ShapeDtypeStruct((n_out, D), jnp.float32),
    )(hs, ha, W_top, W_bot, b.reshape(1, D))


def kernel(x, nodes, neigh_idx, W, b):
    B, n_neigh = neigh_idx.shape
    assert n_neigh == NN and x.shape[1] == D
    pad = B_PAD - B
    nodes_p = jnp.concatenate([nodes, jnp.zeros((pad,), jnp.int32)])
    neigh_p = jnp.concatenate([neigh_idx, jnp.zeros((pad, NN), jnp.int32)], axis=0)
    neigh2 = neigh_p.reshape(B_PAD // G, G * NN)
    hs, ha = _gather_mean(x, nodes_p, neigh2)
    return _linear(hs, ha, W[:D], W[D:] * jnp.float32(1.0 / NN), b, B)


# speedup vs baseline: 1.8240x; 1.0188x over previous
import functools

import jax
import jax.numpy as jnp
from jax import lax
from jax.experimental import pallas as pl
from jax.experimental.pallas import tpu as pltpu
from jax.experimental.pallas import tpu_sc as plsc

NC = 2
NS = 16
L = 16

D = 128
NN = 32
G = 4
ROWS = G * NN
NBUF = 4

FAST_C = 0
F_SEEDS = 608
S_SEEDS = 32
B_PAD = NS * (F_SEEDS + S_SEEDS)


def _gather_mean(x, nodes_p, neigh2):
    fg = F_SEEDS // G
    sg = S_SEEDS // G
    mesh = plsc.VectorSubcoreMesh(core_axis_name="c", subcore_axis_name="s")

    @functools.partial(
        pl.kernel,
        mesh=mesh,
        out_type=[
            jax.ShapeDtypeStruct((B_PAD, D), jnp.float32),
            jax.ShapeDtypeStruct((B_PAD, D), jnp.float32),
        ],
        scratch_types=[
            pltpu.VMEM((fg, ROWS), jnp.int32),
            pltpu.VMEM((F_SEEDS,), jnp.int32),
            pltpu.VMEM((128, D), jnp.float32),
            pltpu.VMEM((NBUF, ROWS, D), jnp.float32),
            pltpu.VMEM((NBUF, G, D), jnp.float32),
            pltpu.SemaphoreType.DMA((NBUF,)),
            pltpu.SemaphoreType.DMA((NBUF,)),
            pltpu.SemaphoreType.DMA,
        ],
    )
    def k(x_hbm, nodes_hbm, neigh_hbm, hs_hbm, ha_hbm,
          nidx_v, sidx_v, sbuf, nbuf, hbuf, gsem, osem, ssem):
        c = lax.axis_index("c")
        s = lax.axis_index("s")
        on_fast = c == FAST_C
        my_seeds = jnp.where(on_fast, F_SEEDS, S_SEEDS)
        base_row = jnp.where(on_fast, s * F_SEEDS, NS * F_SEEDS + s * S_SEEDS)
        base_row = pl.multiple_of(base_row, 32)
        n_groups = jnp.where(on_fast, fg, sg)
        gbase = base_row // G
        gbase = pl.multiple_of(gbase, 8)

        @pl.when(on_fast)
        def _():
            pltpu.sync_copy(neigh_hbm.at[pl.ds(gbase, fg)], nidx_v)
            pltpu.sync_copy(nodes_hbm.at[pl.ds(base_row, F_SEEDS)], sidx_v)

        @pl.when(jnp.logical_not(on_fast))
        def _():
            pltpu.sync_copy(
                neigh_hbm.at[pl.ds(gbase, sg)], nidx_v.at[pl.ds(0, sg)]
            )
            pltpu.sync_copy(
                nodes_hbm.at[pl.ds(base_row, S_SEEDS)],
                sidx_v.at[pl.ds(0, S_SEEDS)],
            )

        def self_chunk(lo, sz):
            pltpu.async_copy(
                x_hbm.at[sidx_v.at[pl.ds(lo, sz)]], sbuf.at[pl.ds(0, sz)], ssem
            ).wait()
            pltpu.sync_copy(
                sbuf.at[pl.ds(0, sz)], hs_hbm.at[pl.ds(base_row + lo, sz)]
            )

        def self_loop(i, carry):
            self_chunk(i * 128, 128)
            return carry

        n_full = my_seeds // 128
        lax.fori_loop(0, n_full, self_loop, 0)
        rem_lo = n_full * 128

        @pl.when(on_fast)
        def _():
            self_chunk(rem_lo, F_SEEDS % 128)

        @pl.when(jnp.logical_not(on_fast))
        def _():
            self_chunk(rem_lo, S_SEEDS)

        def gather(g, slot):
            return pltpu.make_async_copy(
                x_hbm.at[nidx_v.at[g]], nbuf.at[slot], gsem.at[slot]
            )

        def agg_write(g, slot):
            return pltpu.make_async_copy(
                hbuf.at[slot],
                ha_hbm.at[pl.ds(base_row + g * G, G)],
                osem.at[slot],
            )

        for slot in range(NBUF):
            gather(slot, slot).start()

        n_outer = n_groups // NBUF

        def outer(go, carry):
            for slot in range(NBUF):
                g = go * NBUF + slot
                gather(g, slot).wait()
                @pl.when(go > 0)
                def _():
                    agg_write(g - NBUF, slot).wait()
                for si in range(G):
                    UNR = 8

                    def body(t, accs):
                        row0 = si * NN + t * UNR
                        for u in range(UNR):
                            accs = tuple(
                                accs[ci] + nbuf[slot, row0 + u, pl.ds(ci * L, L)]
                                for ci in range(D // L)
                            )
                        return accs

                    accs = lax.fori_loop(
                        0, NN // UNR, body,
                        tuple(jnp.zeros((L,), jnp.float32) for _ in range(D // L)),
                    )
                    for ci in range(D // L):
                        hbuf[slot, si, pl.ds(ci * L, L)] = accs[ci]
                agg_write(g, slot).start()
                @pl.when(go < n_outer - 1)
                def _():
                    gather(g + NBUF, slot).start()
            return carry

        lax.fori_loop(0, n_outer, outer, 0)
        for slot in range(NBUF):
            agg_write((n_outer - 1) * NBUF + slot, slot).wait()

    return k(x, nodes_p, neigh2)


def _mm_body(hs_ref, ha_ref, wt_ref, wb_ref, b_ref, o_ref):
    dims = (((1,), (0,)), ((), ()))
    o_ref[...] = (
        lax.dot_general(hs_ref[...], wt_ref[...], dims,
                        preferred_element_type=jnp.float32)
        + lax.dot_general(ha_ref[...], wb_ref[...], dims,
                          preferred_element_type=jnp.float32)
        + b_ref[...]
    )


def _linear(hs, ha, W_top, W_bot, b, n_out):
    blk = 1024
    grid = hs.shape[0] // blk
    return pl.pallas_call(
        _mm_body,
        grid=(grid,),
        in_specs=[
            pl.BlockSpec((blk, D), lambda i: (i, 0)),
            pl.BlockSpec((blk, D), lambda i: (i, 0)),
            pl.BlockSpec((D, D), lambda i: (0, 0)),
            pl.BlockSpec((D, D), lambda i: (0, 0)),
            pl.BlockSpec((1, D), lambda i: (0, 0)),
        ],
        out_specs=pl.BlockSpec((blk, D), lambda i: (i, 0)),
        out_shape=jax.ShapeDtypeStruct((n_out, D), jnp.float32),
    )(hs, ha, W_top, W_bot, b.reshape(1, D))


def kernel(x, nodes, neigh_idx, W, b):
    B, n_neigh = neigh_idx.shape
    assert n_neigh == NN and x.shape[1] == D
    pad = B_PAD - B
    nodes_p = jnp.concatenate([nodes, jnp.zeros((pad,), jnp.int32)])
    neigh_p = jnp.concatenate([neigh_idx, jnp.zeros((pad, NN), jnp.int32)], axis=0)
    neigh2 = neigh_p.reshape(B_PAD // G, G * NN)
    hs, ha = _gather_mean(x, nodes_p, neigh2)
    return _linear(hs, ha, W[:D], W[D:] * jnp.float32(1.0 / NN), b, B)
